# baseline (speedup 1.0000x reference)
import functools

import jax
import jax.numpy as jnp
from jax import lax
from jax.experimental import pallas as pl
from jax.experimental.pallas import tpu as pltpu
from jax.experimental.pallas import tpu_sc as plsc

NUM_WORKERS = 32
LANES = 16
UNROLL = 8
NBUF = 8
BLK = 32768


def _project_kernel(wt_ref, tt_ref, o_ref, *, inv_len):
    w = wt_ref[...] * inv_len
    o_ref[...] = jnp.dot(w, tt_ref[...], preferred_element_type=jnp.float32)


def _project(tt, wt_pad, seq_len):
    k, v = tt.shape
    grid = (v + BLK - 1) // BLK
    return pl.pallas_call(
        functools.partial(_project_kernel, inv_len=1.0 / seq_len),
        grid=(grid,),
        in_specs=[
            pl.BlockSpec((LANES, k), lambda i: (0, 0)),
            pl.BlockSpec((k, BLK), lambda i: (0, i)),
        ],
        out_specs=pl.BlockSpec((LANES, BLK), lambda i: (0, i)),
        out_shape=jax.ShapeDtypeStruct((LANES, v), jnp.float32),
    )(wt_pad, tt)


def _gather_logits_kernel(batch, seq_len):
    ex_per_w = batch // NUM_WORKERS

    mesh = plsc.VectorSubcoreMesh(core_axis_name="c", subcore_axis_name="s")

    @functools.partial(
        pl.kernel,
        out_type=jax.ShapeDtypeStruct((batch, LANES), jnp.float32),
        mesh=mesh,
        scratch_types=[
            pltpu.VMEM((ex_per_w, seq_len), jnp.int32),
            [pltpu.VMEM((seq_len, LANES), jnp.float32) for _ in range(NBUF)],
            pltpu.VMEM((ex_per_w, LANES), jnp.float32),
            pltpu.VMEM((LANES,), jnp.float32),
            [pltpu.SemaphoreType.DMA for _ in range(NBUF)],
        ],
        compiler_params=pltpu.CompilerParams(use_tc_tiling_on_sc=False),
    )
    def body(ids_hbm, tp_hbm, b_hbm, out_hbm, idx_v, rows, acc_v, b_v, sems):
        wid = lax.axis_index("s") * 2 + lax.axis_index("c")
        base = wid * ex_per_w
        pltpu.sync_copy(ids_hbm.at[pl.ds(base, ex_per_w)], idx_v)
        pltpu.sync_copy(b_hbm, b_v)

        def start(e, b):
            pltpu.async_copy(tp_hbm.at[idx_v.at[e]], rows[b], sems[b])

        def reduce_rows(r):
            def tbody(t, c):
                tb = t * UNROLL
                for k in range(UNROLL):
                    c = c + r[tb + k, :]
                return c

            zero = jnp.zeros((LANES,), jnp.float32)
            return lax.fori_loop(0, seq_len // UNROLL, tbody, zero)

        for b in range(NBUF):
            start(b, b)

        bias = b_v[:]

        def gbody(g, carry):
            e0 = g * NBUF
            for b in range(NBUF):
                e = e0 + b
                pltpu.make_async_copy(
                    tp_hbm.at[pl.ds(0, seq_len)], rows[b], sems[b]
                ).wait()
                acc_v[e, :] = reduce_rows(rows[b]) + bias

                @pl.when(e + NBUF < ex_per_w)
                def _():
                    start(e + NBUF, b)

            return carry

        lax.fori_loop(0, ex_per_w // NBUF, gbody, 0)
        pltpu.sync_copy(acc_v, out_hbm.at[pl.ds(base, ex_per_w)])

    return body


def kernel(subword_ids, table, W, b):
    batch, seq_len = subword_ids.shape
    embed_dim = table.shape[1]
    num_classes = W.shape[1]

    tt = table.T
    wt_pad = jnp.pad(W.T, ((0, LANES - num_classes), (0, 0)))
    b_pad = jnp.pad(b, (0, LANES - num_classes))

    proj_t = _project(tt, wt_pad, seq_len)
    proj = proj_t.T

    logits_pad = _gather_logits_kernel(batch, seq_len)(
        subword_ids, proj, b_pad
    )
    return logits_pad[:, :num_classes]


# speedup vs baseline: 2.1532x; 2.1532x over previous
import functools

import jax
import jax.numpy as jnp
from jax import lax
from jax.experimental import pallas as pl
from jax.experimental.pallas import tpu as pltpu
from jax.experimental.pallas import tpu_sc as plsc

NUM_WORKERS = 32
LANES = 16
UNROLL = 8
NBUF = 8
BLK = 32768


def _project_kernel(wt_ref, tt_ref, o_ref, *, inv_len):
    w = wt_ref[...] * inv_len
    rt = lax.dot_general(
        tt_ref[...], w, (((0,), (1,)), ((), ())),
        preferred_element_type=jnp.float32,
    )
    b8 = BLK // 8
    o_ref[...] = jnp.concatenate(
        [rt[k * b8:(k + 1) * b8, :] for k in range(8)], axis=1
    )


def _project(tt, wt_pad, seq_len):
    k, v = tt.shape
    grid = (v + BLK - 1) // BLK
    return pl.pallas_call(
        functools.partial(_project_kernel, inv_len=1.0 / seq_len),
        grid=(grid,),
        in_specs=[
            pl.BlockSpec((LANES, k), lambda i: (0, 0)),
            pl.BlockSpec((k, BLK), lambda i: (0, i)),
        ],
        out_specs=pl.BlockSpec((BLK // 8, 8 * LANES), lambda i: (i, 0)),
        out_shape=jax.ShapeDtypeStruct((grid * (BLK // 8), 8 * LANES),
                                       jnp.float32),
        compiler_params=pltpu.CompilerParams(
            vmem_limit_bytes=100 * 1024 * 1024,
            fuse_transposed_lhs_in_matmul=True,
        ),
    )(wt_pad, tt)


def _gather_logits_kernel(batch, seq_len):
    ex_per_w = batch // NUM_WORKERS

    mesh = plsc.VectorSubcoreMesh(core_axis_name="c", subcore_axis_name="s")

    @functools.partial(
        pl.kernel,
        out_type=jax.ShapeDtypeStruct((batch, LANES), jnp.float32),
        mesh=mesh,
        scratch_types=[
            pltpu.VMEM((ex_per_w, seq_len), jnp.int32),
            [pltpu.VMEM((seq_len, LANES), jnp.float32) for _ in range(NBUF)],
            pltpu.VMEM((ex_per_w, LANES), jnp.float32),
            pltpu.VMEM((LANES,), jnp.float32),
            [pltpu.SemaphoreType.DMA for _ in range(NBUF)],
        ],
        compiler_params=pltpu.CompilerParams(use_tc_tiling_on_sc=False),
    )
    def body(ids_hbm, tp_hbm, b_hbm, out_hbm, idx_v, rows, acc_v, b_v, sems):
        wid = lax.axis_index("s") * 2 + lax.axis_index("c")
        base = wid * ex_per_w
        pltpu.sync_copy(ids_hbm.at[pl.ds(base, ex_per_w)], idx_v)
        pltpu.sync_copy(b_hbm, b_v)

        def start(e, b):
            pltpu.async_copy(tp_hbm.at[idx_v.at[e]], rows[b], sems[b])

        def reduce_rows(r):
            def tbody(t, c):
                tb = t * UNROLL
                for k in range(UNROLL):
                    c = c + r[tb + k, :]
                return c

            zero = jnp.zeros((LANES,), jnp.float32)
            return lax.fori_loop(0, seq_len // UNROLL, tbody, zero)

        for b in range(NBUF):
            start(b, b)

        bias = b_v[:]

        def gbody(g, carry):
            e0 = g * NBUF
            for b in range(NBUF):
                e = e0 + b
                pltpu.make_async_copy(
                    tp_hbm.at[pl.ds(0, seq_len)], rows[b], sems[b]
                ).wait()
                acc_v[e, :] = reduce_rows(rows[b]) + bias

                @pl.when(e + NBUF < ex_per_w)
                def _():
                    start(e + NBUF, b)

            return carry

        lax.fori_loop(0, ex_per_w // NBUF, gbody, 0)
        pltpu.sync_copy(acc_v, out_hbm.at[pl.ds(base, ex_per_w)])

    return body


def kernel(subword_ids, table, W, b):
    batch, seq_len = subword_ids.shape
    embed_dim = table.shape[1]
    num_classes = W.shape[1]

    tt = table.T
    wt_pad = jnp.pad(W.T, ((0, LANES - num_classes), (0, 0)))
    b_pad = jnp.pad(b, (0, LANES - num_classes))

    packed = _project(tt, wt_pad, seq_len)
    proj = packed.reshape(-1, LANES)

    t = subword_ids & (BLK - 1)
    ids_rho = (subword_ids - t) + ((t & (BLK // 8 - 1)) << 3) + (t >> 12)

    logits_pad = _gather_logits_kernel(batch, seq_len)(
        ids_rho, proj, b_pad
    )
    return logits_pad[:, :num_classes]


# speedup vs baseline: 2.1933x; 1.0186x over previous
import functools

import jax
import jax.numpy as jnp
from jax import lax
from jax.experimental import pallas as pl
from jax.experimental.pallas import tpu as pltpu
from jax.experimental.pallas import tpu_sc as plsc

NUM_WORKERS = 32
LANES = 16
UNROLL = 8
NBUF = 8
BLK = 32768


def _project_kernel(wt_ref, tt_ref, o_ref, *, inv_len):
    w = wt_ref[...] * inv_len
    rt = lax.dot_general(
        tt_ref[...], w, (((0,), (1,)), ((), ())),
        preferred_element_type=jnp.float32,
    )
    b8 = BLK // 8
    for k in range(8):
        o_ref[:, k * LANES:(k + 1) * LANES] = rt[k * b8:(k + 1) * b8, :]


def _project(tt, wt_pad, seq_len):
    k, v = tt.shape
    grid = (v + BLK - 1) // BLK
    return pl.pallas_call(
        functools.partial(_project_kernel, inv_len=1.0 / seq_len),
        grid=(grid,),
        in_specs=[
            pl.BlockSpec((LANES, k), lambda i: (0, 0)),
            pl.BlockSpec((k, BLK), lambda i: (0, i)),
        ],
        out_specs=pl.BlockSpec((BLK // 8, 8 * LANES), lambda i: (i, 0)),
        out_shape=jax.ShapeDtypeStruct((grid * (BLK // 8), 8 * LANES),
                                       jnp.float32),
        compiler_params=pltpu.CompilerParams(
            vmem_limit_bytes=100 * 1024 * 1024,
            fuse_transposed_lhs_in_matmul=True,
        ),
    )(wt_pad, tt)


def _gather_logits_kernel(batch, seq_len):
    ex_per_w = batch // NUM_WORKERS

    mesh = plsc.VectorSubcoreMesh(core_axis_name="c", subcore_axis_name="s")

    @functools.partial(
        pl.kernel,
        out_type=jax.ShapeDtypeStruct((batch, LANES), jnp.float32),
        mesh=mesh,
        scratch_types=[
            pltpu.VMEM((ex_per_w, seq_len), jnp.int32),
            [pltpu.VMEM((seq_len, LANES), jnp.float32) for _ in range(NBUF)],
            pltpu.VMEM((ex_per_w, LANES), jnp.float32),
            pltpu.VMEM((LANES,), jnp.float32),
            [pltpu.SemaphoreType.DMA for _ in range(NBUF)],
        ],
        compiler_params=pltpu.CompilerParams(use_tc_tiling_on_sc=False),
    )
    def body(ids_hbm, tp_hbm, b_hbm, out_hbm, idx_v, rows, acc_v, b_v, sems):
        wid = lax.axis_index("s") * 2 + lax.axis_index("c")
        base = wid * ex_per_w
        pltpu.sync_copy(ids_hbm.at[pl.ds(base, ex_per_w)], idx_v)
        pltpu.sync_copy(b_hbm, b_v)

        def start(e, b):
            pltpu.async_copy(tp_hbm.at[idx_v.at[e]], rows[b], sems[b])

        def reduce_rows(r):
            def tbody(t, c):
                tb = t * UNROLL
                for k in range(UNROLL):
                    c = c + r[tb + k, :]
                return c

            zero = jnp.zeros((LANES,), jnp.float32)
            return lax.fori_loop(0, seq_len // UNROLL, tbody, zero)

        for b in range(NBUF):
            start(b, b)

        bias = b_v[:]

        def gbody(g, carry):
            e0 = g * NBUF
            for b in range(NBUF):
                e = e0 + b
                pltpu.make_async_copy(
                    tp_hbm.at[pl.ds(0, seq_len)], rows[b], sems[b]
                ).wait()
                acc_v[e, :] = reduce_rows(rows[b]) + bias

                @pl.when(e + NBUF < ex_per_w)
                def _():
                    start(e + NBUF, b)

            return carry

        lax.fori_loop(0, ex_per_w // NBUF, gbody, 0)
        pltpu.sync_copy(acc_v, out_hbm.at[pl.ds(base, ex_per_w)])

    return body


def kernel(subword_ids, table, W, b):
    batch, seq_len = subword_ids.shape
    embed_dim = table.shape[1]
    num_classes = W.shape[1]

    tt = table.T
    wt_pad = jnp.pad(W.T, ((0, LANES - num_classes), (0, 0)))
    b_pad = jnp.pad(b, (0, LANES - num_classes))

    packed = _project(tt, wt_pad, seq_len)
    proj = packed.reshape(-1, LANES)

    t = subword_ids & (BLK - 1)
    ids_rho = (subword_ids - t) + ((t & (BLK // 8 - 1)) << 3) + (t >> 12)

    logits_pad = _gather_logits_kernel(batch, seq_len)(
        ids_rho, proj, b_pad
    )
    return logits_pad[:, :num_classes]


# speedup vs baseline: 3.6466x; 1.6626x over previous
import functools

import jax
import jax.numpy as jnp
from jax import lax
from jax.experimental import pallas as pl
from jax.experimental.pallas import tpu as pltpu
from jax.experimental.pallas import tpu_sc as plsc

NUM_WORKERS = 32
LANES = 16
UNROLL = 8
NBUF = 8
BLK = 32768


def _project_kernel(wbd_ref, tt_ref, o_ref):
    b8 = BLK // 8
    a = jnp.concatenate(
        [
            jnp.transpose(tt_ref[:, k * b8:(k + 1) * b8]).astype(jnp.bfloat16)
            for k in range(8)
        ],
        axis=1,
    )
    o_ref[...] = jnp.dot(a, wbd_ref[...],
                         preferred_element_type=jnp.float32)


def _project(tt, wbd, seq_len):
    k, v = tt.shape
    grid = (v + BLK - 1) // BLK
    return pl.pallas_call(
        _project_kernel,
        grid=(grid,),
        in_specs=[
            pl.BlockSpec((8 * k, 8 * LANES), lambda i: (0, 0)),
            pl.BlockSpec((k, BLK), lambda i: (0, i)),
        ],
        out_specs=pl.BlockSpec((BLK // 8, 8 * LANES), lambda i: (i, 0)),
        out_shape=jax.ShapeDtypeStruct((grid * (BLK // 8), 8 * LANES),
                                       jnp.float32),
        compiler_params=pltpu.CompilerParams(
            vmem_limit_bytes=100 * 1024 * 1024,
        ),
    )(wbd, tt)


def _gather_logits_kernel(batch, seq_len):
    ex_per_w = batch // NUM_WORKERS

    mesh = plsc.VectorSubcoreMesh(core_axis_name="c", subcore_axis_name="s")

    @functools.partial(
        pl.kernel,
        out_type=jax.ShapeDtypeStruct((batch, LANES), jnp.float32),
        mesh=mesh,
        scratch_types=[
            pltpu.VMEM((ex_per_w, seq_len), jnp.int32),
            [pltpu.VMEM((seq_len, LANES), jnp.float32) for _ in range(NBUF)],
            pltpu.VMEM((ex_per_w, LANES), jnp.float32),
            pltpu.VMEM((LANES,), jnp.float32),
            [pltpu.SemaphoreType.DMA for _ in range(NBUF)],
        ],
        compiler_params=pltpu.CompilerParams(use_tc_tiling_on_sc=False),
    )
    def body(ids_hbm, tp_hbm, b_hbm, out_hbm, idx_v, rows, acc_v, b_v, sems):
        wid = lax.axis_index("s") * 2 + lax.axis_index("c")
        base = wid * ex_per_w
        pltpu.sync_copy(ids_hbm.at[pl.ds(base, ex_per_w)], idx_v)
        pltpu.sync_copy(b_hbm, b_v)

        def start(e, b):
            pltpu.async_copy(tp_hbm.at[idx_v.at[e]], rows[b], sems[b])

        def reduce_rows(r):
            def tbody(t, c):
                tb = t * UNROLL
                for k in range(UNROLL):
                    c = c + r[tb + k, :]
                return c

            zero = jnp.zeros((LANES,), jnp.float32)
            return lax.fori_loop(0, seq_len // UNROLL, tbody, zero)

        for b in range(NBUF):
            start(b, b)

        bias = b_v[:]

        def gbody(g, carry):
            e0 = g * NBUF
            for b in range(NBUF):
                e = e0 + b
                pltpu.make_async_copy(
                    tp_hbm.at[pl.ds(0, seq_len)], rows[b], sems[b]
                ).wait()
                acc_v[e, :] = reduce_rows(rows[b]) + bias

                @pl.when(e + NBUF < ex_per_w)
                def _():
                    start(e + NBUF, b)

            return carry

        lax.fori_loop(0, ex_per_w // NBUF, gbody, 0)
        pltpu.sync_copy(acc_v, out_hbm.at[pl.ds(base, ex_per_w)])

    return body


def kernel(subword_ids, table, W, b):
    batch, seq_len = subword_ids.shape
    embed_dim = table.shape[1]
    num_classes = W.shape[1]

    tt = table.T
    b_pad = jnp.pad(b, (0, LANES - num_classes))
    w16 = jnp.pad(W, ((0, 0), (0, LANES - num_classes))) / seq_len
    wbd = (
        jnp.eye(8, dtype=jnp.float32)[:, None, :, None]
        * w16[None, :, None, :]
    ).reshape(8 * embed_dim, 8 * LANES).astype(jnp.bfloat16)

    packed = _project(tt, wbd, seq_len)
    proj = packed.reshape(-1, LANES)

    t = subword_ids & (BLK - 1)
    ids_rho = (subword_ids - t) + ((t & (BLK // 8 - 1)) << 3) + (t >> 12)

    logits_pad = _gather_logits_kernel(batch, seq_len)(
        ids_rho, proj, b_pad
    )
    return logits_pad[:, :num_classes]


# speedup vs baseline: 3.7413x; 1.0260x over previous
import functools

import jax
import jax.numpy as jnp
from jax import lax
from jax.experimental import pallas as pl
from jax.experimental.pallas import tpu as pltpu
from jax.experimental.pallas import tpu_sc as plsc

NUM_WORKERS = 32
LANES = 16
UNROLL = 8
NBUF = 8
BLK = 65536


def _project_kernel(wbd_ref, tt_ref, o_ref):
    b8 = BLK // 8
    a = jnp.concatenate(
        [
            jnp.transpose(tt_ref[:, k * b8:(k + 1) * b8]).astype(jnp.bfloat16)
            for k in range(8)
        ],
        axis=1,
    )
    o_ref[...] = jnp.dot(a, wbd_ref[...],
                         preferred_element_type=jnp.float32)


def _project(tt, wbd, seq_len):
    k, v = tt.shape
    grid = (v + BLK - 1) // BLK
    return pl.pallas_call(
        _project_kernel,
        grid=(grid,),
        in_specs=[
            pl.BlockSpec((8 * k, 8 * LANES), lambda i: (0, 0)),
            pl.BlockSpec((k, BLK), lambda i: (0, i)),
        ],
        out_specs=pl.BlockSpec((BLK // 8, 8 * LANES), lambda i: (i, 0)),
        out_shape=jax.ShapeDtypeStruct((grid * (BLK // 8), 8 * LANES),
                                       jnp.float32),
        compiler_params=pltpu.CompilerParams(
            vmem_limit_bytes=100 * 1024 * 1024,
        ),
    )(wbd, tt)


def _gather_logits_kernel(batch, seq_len):
    ex_per_w = batch // NUM_WORKERS

    mesh = plsc.VectorSubcoreMesh(core_axis_name="c", subcore_axis_name="s")

    @functools.partial(
        pl.kernel,
        out_type=jax.ShapeDtypeStruct((batch, LANES), jnp.float32),
        mesh=mesh,
        scratch_types=[
            pltpu.VMEM((ex_per_w, seq_len), jnp.int32),
            [pltpu.VMEM((seq_len, LANES), jnp.float32) for _ in range(NBUF)],
            pltpu.VMEM((ex_per_w, LANES), jnp.float32),
            pltpu.VMEM((LANES,), jnp.float32),
            [pltpu.SemaphoreType.DMA for _ in range(NBUF)],
        ],
        compiler_params=pltpu.CompilerParams(use_tc_tiling_on_sc=False),
    )
    def body(ids_hbm, tp_hbm, b_hbm, out_hbm, idx_v, rows, acc_v, b_v, sems):
        wid = lax.axis_index("s") * 2 + lax.axis_index("c")
        base = wid * ex_per_w
        pltpu.sync_copy(ids_hbm.at[pl.ds(base, ex_per_w)], idx_v)
        pltpu.sync_copy(b_hbm, b_v)

        def start(e, b):
            pltpu.async_copy(tp_hbm.at[idx_v.at[e]], rows[b], sems[b])

        def reduce_rows(r):
            def tbody(t, c):
                tb = t * UNROLL
                for k in range(UNROLL):
                    c = c + r[tb + k, :]
                return c

            zero = jnp.zeros((LANES,), jnp.float32)
            return lax.fori_loop(0, seq_len // UNROLL, tbody, zero)

        for b in range(NBUF):
            start(b, b)

        bias = b_v[:]

        def gbody(g, carry):
            e0 = g * NBUF
            for b in range(NBUF):
                e = e0 + b
                pltpu.make_async_copy(
                    tp_hbm.at[pl.ds(0, seq_len)], rows[b], sems[b]
                ).wait()
                acc_v[e, :] = reduce_rows(rows[b]) + bias

                @pl.when(e + NBUF < ex_per_w)
                def _():
                    start(e + NBUF, b)

            return carry

        lax.fori_loop(0, ex_per_w // NBUF, gbody, 0)
        pltpu.sync_copy(acc_v, out_hbm.at[pl.ds(base, ex_per_w)])

    return body


def kernel(subword_ids, table, W, b):
    batch, seq_len = subword_ids.shape
    embed_dim = table.shape[1]
    num_classes = W.shape[1]

    tt = table.T
    b_pad = jnp.pad(b, (0, LANES - num_classes))
    w16 = jnp.pad(W, ((0, 0), (0, LANES - num_classes))) / seq_len
    wbd = (
        jnp.eye(8, dtype=jnp.float32)[:, None, :, None]
        * w16[None, :, None, :]
    ).reshape(8 * embed_dim, 8 * LANES).astype(jnp.bfloat16)

    packed = _project(tt, wbd, seq_len)
    proj = packed.reshape(-1, LANES)

    t = subword_ids & (BLK - 1)
    ids_rho = (subword_ids - t) + ((t & (BLK // 8 - 1)) << 3) + (t >> 13)

    logits_pad = _gather_logits_kernel(batch, seq_len)(
        ids_rho, proj, b_pad
    )
    return logits_pad[:, :num_classes]
